# baseline (speedup 1.0000x reference)
import functools

import jax
import jax.numpy as jnp
from jax import lax
from jax.experimental import pallas as pl
from jax.experimental.pallas import tpu as pltpu
from jax.experimental.pallas import tpu_sc as plsc

EPS = 1e-6
LANES = 16
CHUNK = 128


def _rsqrt(x):
    xi = lax.bitcast_convert_type(x, jnp.int32)
    yi = jnp.int32(0x5F3759DF) - (xi >> 1)
    y = lax.bitcast_convert_type(yi, jnp.float32)
    for _ in range(2):
        y = y * (1.5 - 0.5 * x * y * y)
    return y


def kernel(src, seg, table, gamma, beta):
    del seg
    B, L = src.shape
    V, E = table.shape
    n_vec = E // LANES
    N = B * L

    info = plsc.get_sparse_core_info()
    NC, NS = info.num_cores, info.num_subcores
    NW = NC * NS
    per_w = N // NW
    n_chunks = per_w // CHUNK
    assert per_w * NW == N and n_chunks * CHUNK == per_w
    assert n_chunks >= 4 and n_chunks % 2 == 0
    assert E == 64 and V % 2 == 0 and CHUNK % LANES == 0

    src_flat = src.reshape(NW, n_chunks, CHUNK)
    table2 = table.reshape(V // 2, 2 * E)
    mesh = plsc.VectorSubcoreMesh(core_axis_name="c", subcore_axis_name="s")

    @functools.partial(
        pl.kernel,
        mesh=mesh,
        out_type=jax.ShapeDtypeStruct((N, E), jnp.float32),
        compiler_params=pltpu.CompilerParams(needs_layout_passes=False),
        scratch_types=[
            pltpu.VMEM((n_chunks, CHUNK), jnp.int32),
            pltpu.VMEM((CHUNK,), jnp.int32),
            pltpu.VMEM((CHUNK,), jnp.int32),
            pltpu.VMEM((CHUNK, 2 * E), jnp.float32),
            pltpu.VMEM((CHUNK, 2 * E), jnp.float32),
            pltpu.VMEM((CHUNK, E), jnp.float32),
            pltpu.VMEM((CHUNK, E), jnp.float32),
            pltpu.VMEM((E,), jnp.float32),
            pltpu.VMEM((E,), jnp.float32),
            pltpu.SemaphoreType.DMA,
            pltpu.SemaphoreType.DMA,
            pltpu.SemaphoreType.DMA,
            pltpu.SemaphoreType.DMA,
        ],
    )
    def emb_ln(table_hbm, idx_hbm, gamma_hbm, beta_hbm, out_hbm,
               idx_v, idx2a, idx2b, rows0, rows1, outv0, outv1,
               gamma_v, beta_v, gsem0, gsem1, osem0, osem1):
        rows_vs = (rows0, rows1)
        out_vs = (outv0, outv1)
        idx2_vs = (idx2a, idx2b)
        gsems = (gsem0, gsem1)
        osems = (osem0, osem1)

        wid = lax.axis_index("s") * NC + lax.axis_index("c")
        base = wid * per_w
        pltpu.sync_copy(idx_hbm.at[wid], idx_v)
        pltpu.sync_copy(gamma_hbm, gamma_v)
        pltpu.sync_copy(beta_hbm, beta_v)
        g = [gamma_v[pl.ds(j * LANES, LANES)] for j in range(n_vec)]
        bta = [beta_v[pl.ds(j * LANES, LANES)] for j in range(n_vec)]
        inv_e = jnp.float32(1.0 / E)
        lane = lax.iota(jnp.int32, LANES)

        def fill_idx2(c, b):
            for j in range(CHUNK // LANES):
                idx2_vs[b][pl.ds(j * LANES, LANES)] = (
                    idx_v[c, pl.ds(j * LANES, LANES)] >> 1)

        def gather_start(c, b):
            pltpu.async_copy(table_hbm.at[idx2_vs[b]], rows_vs[b], gsems[b])

        def gather_wait(c, b):
            pltpu.make_async_copy(
                table_hbm.at[idx2_vs[b]], rows_vs[b], gsems[b]).wait()

        def out_start(c, b):
            start = pl.multiple_of(base + c * CHUNK, 8)
            pltpu.async_copy(
                out_vs[b], out_hbm.at[pl.ds(start, CHUNK)], osems[b])

        def out_wait(c, b):
            start = pl.multiple_of(base + c * CHUNK, 8)
            pltpu.make_async_copy(
                out_vs[b], out_hbm.at[pl.ds(start, CHUNK)], osems[b]).wait()

        def compute(c, b):
            rows_v = rows_vs[b]
            out_v = out_vs[b]

            def group_body(gi, _):
                g0 = gi * LANES
                row_ids = g0 + lane
                rawv = idx_v[c, pl.ds(g0, LANES)]
                col0 = (rawv & 1) << 6
                nacc = 4
                s = [jnp.zeros((LANES,), jnp.float32) for _ in range(nacc)]
                q = [jnp.zeros((LANES,), jnp.float32) for _ in range(nacc)]
                for j in range(E):
                    v = plsc.load_gather(rows_v, [row_ids, col0 + j])
                    s[j % nacc] = s[j % nacc] + v
                    q[j % nacc] = q[j % nacc] + v * v
                mean = ((s[0] + s[1]) + (s[2] + s[3])) * inv_e
                var = ((q[0] + q[1]) + (q[2] + q[3])) * inv_e - mean * mean
                inv = _rsqrt(var + EPS)
                for j in range(E):
                    v = plsc.load_gather(rows_v, [row_ids, col0 + j])
                    sel = jnp.full((LANES,), j % LANES, jnp.int32)
                    gj = g[j // LANES].at[sel].get(mode="promise_in_bounds")
                    bj = bta[j // LANES].at[sel].get(mode="promise_in_bounds")
                    o = (v - mean) * inv * gj + bj
                    plsc.store_scatter(
                        out_v, [row_ids, jnp.full((LANES,), j, jnp.int32)], o)
                return 0

            lax.fori_loop(0, CHUNK // LANES, group_body, 0)

        def stage(c0, b):
            c = c0 + b
            gather_wait(c, b)
            pl.when(c >= 2)(lambda: out_wait(c - 2, b))
            compute(c, b)
            out_start(c, b)

            @pl.when(c + 2 < n_chunks)
            def _():
                fill_idx2(c + 2, b)
                gather_start(c + 2, b)

        fill_idx2(0, 0)
        gather_start(0, 0)
        fill_idx2(1, 1)
        gather_start(1, 1)

        def loop_body(i, _):
            c0 = 2 * i
            stage(c0, 0)
            stage(c0, 1)
            return 0

        lax.fori_loop(0, n_chunks // 2, loop_body, 0)
        out_wait(n_chunks - 2, 0)
        out_wait(n_chunks - 1, 1)

    out = emb_ln(table2, src_flat, gamma, beta)
    return out.reshape(B, L, E)


# speedup vs baseline: 1.7914x; 1.7914x over previous
import functools

import jax
import jax.numpy as jnp
from jax import lax
from jax.experimental import pallas as pl
from jax.experimental.pallas import tpu as pltpu
from jax.experimental.pallas import tpu_sc as plsc

EPS = 1e-6
LANES = 16
CHUNK = 128


def _rsqrt(x):
    xi = lax.bitcast_convert_type(x, jnp.int32)
    yi = jnp.int32(0x5F3759DF) - (xi >> 1)
    y = lax.bitcast_convert_type(yi, jnp.float32)
    for _ in range(2):
        y = y * (1.5 - 0.5 * x * y * y)
    return y


def kernel(src, seg, table, gamma, beta):
    del seg
    B, L = src.shape
    V, E = table.shape
    n_vec = E // LANES
    N = B * L

    info = plsc.get_sparse_core_info()
    NC, NS = info.num_cores, info.num_subcores
    NW = NC * NS
    per_w = N // NW
    n_chunks = per_w // CHUNK
    assert per_w * NW == N and n_chunks * CHUNK == per_w
    assert n_chunks >= 4 and n_chunks % 2 == 0
    assert E == 64 and V % 2 == 0 and CHUNK % LANES == 0

    src_flat = src.reshape(NW, n_chunks, CHUNK)
    table2 = table.reshape(V // 2, 2 * E)
    mesh = plsc.VectorSubcoreMesh(core_axis_name="c", subcore_axis_name="s")

    @functools.partial(
        pl.kernel,
        mesh=mesh,
        out_type=jax.ShapeDtypeStruct((N, E), jnp.float32),
        compiler_params=pltpu.CompilerParams(needs_layout_passes=False),
        scratch_types=[
            pltpu.VMEM((n_chunks, CHUNK), jnp.int32),
            pltpu.VMEM((CHUNK,), jnp.int32),
            pltpu.VMEM((CHUNK,), jnp.int32),
            pltpu.VMEM((CHUNK, 2 * E), jnp.float32),
            pltpu.VMEM((CHUNK, 2 * E), jnp.float32),
            pltpu.VMEM((CHUNK, E), jnp.float32),
            pltpu.VMEM((CHUNK, E), jnp.float32),
            pltpu.VMEM((LANES, LANES), jnp.float32),
            pltpu.VMEM((LANES, LANES), jnp.float32),
            pltpu.VMEM((LANES, E), jnp.float32),
            pltpu.VMEM((E,), jnp.float32),
            pltpu.VMEM((E,), jnp.float32),
            pltpu.SemaphoreType.DMA,
            pltpu.SemaphoreType.DMA,
            pltpu.SemaphoreType.DMA,
            pltpu.SemaphoreType.DMA,
        ],
    )
    def emb_ln(table_hbm, idx_hbm, gamma_hbm, beta_hbm, out_hbm,
               idx_v, idx2a, idx2b, rows0, rows1, outv0, outv1,
               sbuf, qbuf, xbuf, gamma_v, beta_v,
               gsem0, gsem1, osem0, osem1):
        rows_vs = (rows0, rows1)
        out_vs = (outv0, outv1)
        idx2_vs = (idx2a, idx2b)
        gsems = (gsem0, gsem1)
        osems = (osem0, osem1)

        wid = lax.axis_index("s") * NC + lax.axis_index("c")
        base = wid * per_w
        pltpu.sync_copy(idx_hbm.at[wid], idx_v)
        pltpu.sync_copy(gamma_hbm, gamma_v)
        pltpu.sync_copy(beta_hbm, beta_v)
        g = [gamma_v[pl.ds(j * LANES, LANES)] for j in range(n_vec)]
        bta = [beta_v[pl.ds(j * LANES, LANES)] for j in range(n_vec)]
        inv_e = jnp.float32(1.0 / E)
        lane = lax.iota(jnp.int32, LANES)

        def fill_idx2(c, b):
            for j in range(CHUNK // LANES):
                idx2_vs[b][pl.ds(j * LANES, LANES)] = (
                    idx_v[c, pl.ds(j * LANES, LANES)] >> 1)

        def gather_start(c, b):
            pltpu.async_copy(table_hbm.at[idx2_vs[b]], rows_vs[b], gsems[b])

        def gather_wait(c, b):
            pltpu.make_async_copy(
                table_hbm.at[idx2_vs[b]], rows_vs[b], gsems[b]).wait()

        def out_start(c, b):
            start = pl.multiple_of(base + c * CHUNK, 8)
            pltpu.async_copy(
                out_vs[b], out_hbm.at[pl.ds(start, CHUNK)], osems[b])

        def out_wait(c, b):
            start = pl.multiple_of(base + c * CHUNK, 8)
            pltpu.make_async_copy(
                out_vs[b], out_hbm.at[pl.ds(start, CHUNK)], osems[b]).wait()

        def compute(c, b):
            rows_v = rows_vs[b]
            out_v = out_vs[b]

            def group_body(gi, _):
                g0 = gi * LANES
                rawv = idx_v[c, pl.ds(g0, LANES)]
                odd = rawv & 1
                for rr in range(LANES):
                    r = g0 + rr
                    sel = jnp.full((LANES,), rr, jnp.int32)
                    m = odd.at[sel].get(mode="promise_in_bounds") > 0
                    x = []
                    for j in range(n_vec):
                        lo = rows_v[r, pl.ds(j * LANES, LANES)]
                        hi = rows_v[r, pl.ds(E + j * LANES, LANES)]
                        x.append(jnp.where(m, hi, lo))
                    s = (x[0] + x[1]) + (x[2] + x[3])
                    q = (x[0] * x[0] + x[1] * x[1]) + (
                        x[2] * x[2] + x[3] * x[3])
                    sbuf[rr, pl.ds(0, LANES)] = s
                    qbuf[rr, pl.ds(0, LANES)] = q
                    for j in range(n_vec):
                        xbuf[rr, pl.ds(j * LANES, LANES)] = x[j]
                svs = [plsc.load_gather(
                    sbuf, [lane, jnp.full((LANES,), l, jnp.int32)])
                    for l in range(LANES)]
                qvs = [plsc.load_gather(
                    qbuf, [lane, jnp.full((LANES,), l, jnp.int32)])
                    for l in range(LANES)]
                while len(svs) > 1:
                    svs = [a + c2 for a, c2 in zip(svs[::2], svs[1::2])]
                while len(qvs) > 1:
                    qvs = [a + c2 for a, c2 in zip(qvs[::2], qvs[1::2])]
                mean = svs[0] * inv_e
                var = qvs[0] * inv_e - mean * mean
                inv = _rsqrt(var + EPS)
                for rr in range(LANES):
                    r = g0 + rr
                    sel = jnp.full((LANES,), rr, jnp.int32)
                    m_r = mean.at[sel].get(mode="promise_in_bounds")
                    i_r = inv.at[sel].get(mode="promise_in_bounds")
                    for j in range(n_vec):
                        xj = xbuf[rr, pl.ds(j * LANES, LANES)]
                        out_v[r, pl.ds(j * LANES, LANES)] = (
                            (xj - m_r) * (i_r * g[j]) + bta[j])
                return 0

            lax.fori_loop(0, CHUNK // LANES, group_body, 0)

        def stage(c0, b):
            c = c0 + b
            gather_wait(c, b)
            pl.when(c >= 2)(lambda: out_wait(c - 2, b))
            compute(c, b)
            out_start(c, b)

            @pl.when(c + 2 < n_chunks)
            def _():
                fill_idx2(c + 2, b)
                gather_start(c + 2, b)

        fill_idx2(0, 0)
        gather_start(0, 0)
        fill_idx2(1, 1)
        gather_start(1, 1)

        def loop_body(i, _):
            c0 = 2 * i
            stage(c0, 0)
            stage(c0, 1)
            return 0

        lax.fori_loop(0, n_chunks // 2, loop_body, 0)
        out_wait(n_chunks - 2, 0)
        out_wait(n_chunks - 1, 1)

    out = emb_ln(table2, src_flat, gamma, beta)
    return out.reshape(B, L, E)


# speedup vs baseline: 2.4990x; 1.3950x over previous
import functools

import jax
import jax.numpy as jnp
from jax import lax
from jax.experimental import pallas as pl
from jax.experimental.pallas import tpu as pltpu
from jax.experimental.pallas import tpu_sc as plsc

EPS = 1e-6
LANES = 16
CHUNK = 128


def _rsqrt(x):
    xi = lax.bitcast_convert_type(x, jnp.int32)
    yi = jnp.int32(0x5F3759DF) - (xi >> 1)
    y = lax.bitcast_convert_type(yi, jnp.float32)
    for _ in range(2):
        y = y * (1.5 - 0.5 * x * y * y)
    return y


def kernel(src, seg, table, gamma, beta):
    del seg
    B, L = src.shape
    V, E = table.shape
    n_vec = E // LANES
    N = B * L

    info = plsc.get_sparse_core_info()
    NC, NS = info.num_cores, info.num_subcores
    NW = NC * NS
    per_w = N // NW
    n_chunks = per_w // CHUNK
    assert per_w * NW == N and n_chunks * CHUNK == per_w
    assert n_chunks >= 4 and n_chunks % 2 == 0

    idx = src.reshape(NW, n_chunks, CHUNK)
    mesh = plsc.VectorSubcoreMesh(core_axis_name="c", subcore_axis_name="s")

    @functools.partial(
        pl.kernel,
        mesh=mesh,
        out_type=jax.ShapeDtypeStruct((N, E), jnp.float32),
        compiler_params=pltpu.CompilerParams(
            use_tc_tiling_on_sc=False, needs_layout_passes=False),
        scratch_types=[
            pltpu.VMEM((n_chunks, CHUNK), jnp.int32),
            pltpu.VMEM((CHUNK, E), jnp.float32),
            pltpu.VMEM((CHUNK, E), jnp.float32),
            pltpu.VMEM((CHUNK, E), jnp.float32),
            pltpu.VMEM((CHUNK, E), jnp.float32),
            pltpu.VMEM((LANES, LANES), jnp.float32),
            pltpu.VMEM((LANES, LANES), jnp.float32),
            pltpu.VMEM((E,), jnp.float32),
            pltpu.VMEM((E,), jnp.float32),
            pltpu.SemaphoreType.DMA,
            pltpu.SemaphoreType.DMA,
            pltpu.SemaphoreType.DMA,
            pltpu.SemaphoreType.DMA,
        ],
    )
    def emb_ln(table_hbm, idx_hbm, gamma_hbm, beta_hbm, out_hbm,
               idx_v, rows0, rows1, outv0, outv1, sbuf, qbuf,
               gamma_v, beta_v, gsem0, gsem1, osem0, osem1):
        rows_vs = (rows0, rows1)
        out_vs = (outv0, outv1)
        gsems = (gsem0, gsem1)
        osems = (osem0, osem1)

        wid = lax.axis_index("s") * NC + lax.axis_index("c")
        base = wid * per_w
        pltpu.sync_copy(idx_hbm.at[wid], idx_v)
        pltpu.sync_copy(gamma_hbm, gamma_v)
        pltpu.sync_copy(beta_hbm, beta_v)
        g = [gamma_v[pl.ds(j * LANES, LANES)] for j in range(n_vec)]
        bta = [beta_v[pl.ds(j * LANES, LANES)] for j in range(n_vec)]
        inv_e = jnp.float32(1.0 / E)
        lane = lax.iota(jnp.int32, LANES)

        def gather_start(c, b):
            pltpu.async_copy(table_hbm.at[idx_v.at[c]], rows_vs[b], gsems[b])

        def gather_wait(c, b):
            pltpu.make_async_copy(
                table_hbm.at[idx_v.at[c]], rows_vs[b], gsems[b]).wait()

        def out_start(c, b):
            pltpu.async_copy(
                out_vs[b], out_hbm.at[pl.ds(base + c * CHUNK, CHUNK)],
                osems[b])

        def out_wait(c, b):
            pltpu.make_async_copy(
                out_vs[b], out_hbm.at[pl.ds(base + c * CHUNK, CHUNK)],
                osems[b]).wait()

        def compute(b):
            rows_v = rows_vs[b]
            out_v = out_vs[b]

            def group_body(gi, _):
                g0 = gi * LANES
                for rr in range(LANES):
                    r = g0 + rr
                    x = [rows_v[r, pl.ds(j * LANES, LANES)]
                         for j in range(n_vec)]
                    s = (x[0] + x[1]) + (x[2] + x[3])
                    q = (x[0] * x[0] + x[1] * x[1]) + (
                        x[2] * x[2] + x[3] * x[3])
                    sbuf[rr, pl.ds(0, LANES)] = s
                    qbuf[rr, pl.ds(0, LANES)] = q
                svs = [plsc.load_gather(
                    sbuf, [lane, jnp.full((LANES,), l, jnp.int32)])
                    for l in range(LANES)]
                qvs = [plsc.load_gather(
                    qbuf, [lane, jnp.full((LANES,), l, jnp.int32)])
                    for l in range(LANES)]
                while len(svs) > 1:
                    svs = [a + c for a, c in zip(svs[::2], svs[1::2])]
                while len(qvs) > 1:
                    qvs = [a + c for a, c in zip(qvs[::2], qvs[1::2])]
                mean = svs[0] * inv_e
                var = qvs[0] * inv_e - mean * mean
                inv = _rsqrt(var + EPS)
                for rr in range(LANES):
                    r = g0 + rr
                    sel = jnp.full((LANES,), rr, jnp.int32)
                    m_r = mean.at[sel].get(mode="promise_in_bounds")
                    i_r = inv.at[sel].get(mode="promise_in_bounds")
                    for j in range(n_vec):
                        xj = rows_v[r, pl.ds(j * LANES, LANES)]
                        out_v[r, pl.ds(j * LANES, LANES)] = (
                            (xj - m_r) * (i_r * g[j]) + bta[j])
                return 0

            lax.fori_loop(0, CHUNK // LANES, group_body, 0)

        def stage(c0, b):
            c = c0 + b
            gather_wait(c, b)
            pl.when(c >= 2)(lambda: out_wait(c - 2, b))
            compute(b)
            out_start(c, b)
            pl.when(c + 2 < n_chunks)(lambda: gather_start(c + 2, b))

        gather_start(0, 0)
        gather_start(1, 1)

        def loop_body(i, _):
            c0 = 2 * i
            stage(c0, 0)
            stage(c0, 1)
            return 0

        lax.fori_loop(0, n_chunks // 2, loop_body, 0)
        out_wait(n_chunks - 2, 0)
        out_wait(n_chunks - 1, 1)

    out = emb_ln(table, idx, gamma, beta)
    return out.reshape(B, L, E)
